# baseline (speedup 1.0000x reference)
import functools

import jax
import jax.numpy as jnp
from jax import lax
from jax.experimental import pallas as pl
from jax.experimental.pallas import tpu as pltpu
from jax.experimental.pallas import tpu_sc as plsc

FG_THRESHOLD = 0.5


def _tc_body(roist_ref, gt_ref, labels_ref, gidx_ref, *, nb, tp, m):
    miota = jax.lax.broadcasted_iota(jnp.int32, (m, tp), 0)
    for b in range(nb):
        g = gt_ref[b]
        rt = roist_ref[b]
        inter = None
        va = None
        vb = None
        for c in range(3):
            blo = rt[1 + c : 2 + c, :]
            bhi = rt[4 + c : 5 + c, :]
            glo = g[:, c : c + 1]
            ghi = g[:, 3 + c : 4 + c]
            d = jnp.maximum(jnp.minimum(bhi, ghi) - jnp.maximum(blo, glo), 0.0)
            inter = d if inter is None else inter * d
            sa = jnp.maximum(bhi - blo, 0.0)
            va = sa if va is None else va * sa
            sb = jnp.maximum(ghi - glo, 0.0)
            vb = sb if vb is None else vb * sb
        union = va + vb - inter
        iou = inter / union
        mx = jnp.max(iou, axis=0, keepdims=True)
        labels_ref[b] = (mx[0] >= FG_THRESHOLD).astype(jnp.int32)
        sel = jnp.where(iou == mx, miota, m)
        gidx_ref[b] = jnp.min(sel, axis=0) + b * m


def _make_sc_gather(total, per, tab_words):
    mesh = plsc.VectorSubcoreMesh(core_axis_name="c", subcore_axis_name="s")
    info = plsc.get_sparse_core_info()
    nc = info.num_cores
    lanes = info.num_lanes

    @functools.partial(
        pl.kernel,
        mesh=mesh,
        out_type=jax.ShapeDtypeStruct((total * 6,), jnp.float32),
        scratch_types=[
            pltpu.VMEM((per,), jnp.int32),
            pltpu.VMEM((per * 6,), jnp.float32),
            pltpu.VMEM((tab_words,), jnp.float32),
        ],
        compiler_params=pltpu.CompilerParams(
            use_tc_tiling_on_sc=False, needs_layout_passes=False
        ),
    )
    def sc_gather(table_hbm, idx_hbm, out_hbm, idx_v, rows_v, tab_v):
        wid = lax.axis_index("s") * nc + lax.axis_index("c")
        base = jnp.minimum(wid * per, total - per)
        pltpu.sync_copy(table_hbm, tab_v)
        pltpu.sync_copy(idx_hbm.at[pl.ds(base, per)], idx_v)
        lane_iota = jax.lax.iota(jnp.int32, lanes)

        def body(i, carry):
            j0 = i * lanes
            addr = idx_v[pl.ds(j0, lanes)] * 6
            dst = lane_iota + j0
            for c in range(6):
                vals = plsc.load_gather(tab_v, [addr + c])
                plsc.store_scatter(rows_v, [dst + c * per], vals)
            return carry

        lax.fori_loop(0, per // lanes, body, 0)
        for c in range(6):
            pltpu.sync_copy(
                rows_v.at[pl.ds(c * per, per)],
                out_hbm.at[pl.ds(c * total + base, per)],
            )

    return sc_gather


def kernel(all_rois, gt_boxes, gt_labels, is_sample):
    nb, n, _ = all_rois.shape
    m = gt_boxes.shape[1]
    tp = 4096
    rois_t = jnp.swapaxes(all_rois, 1, 2)
    labels, gidx = pl.pallas_call(
        functools.partial(_tc_body, nb=nb, tp=tp, m=m),
        grid=(pl.cdiv(n, tp),),
        in_specs=[
            pl.BlockSpec((nb, 7, tp), lambda i: (0, 0, i)),
            pl.BlockSpec((nb, m, 6), lambda i: (0, 0, 0)),
        ],
        out_specs=[
            pl.BlockSpec((nb, tp), lambda i: (0, i)),
            pl.BlockSpec((nb, tp), lambda i: (0, i)),
        ],
        out_shape=[
            jax.ShapeDtypeStruct((nb, n), jnp.int32),
            jax.ShapeDtypeStruct((nb, n), jnp.int32),
        ],
        compiler_params=pltpu.CompilerParams(
            dimension_semantics=("arbitrary",),
        ),
    )(rois_t, gt_boxes)

    total = nb * n
    info = plsc.get_sparse_core_info()
    nw = info.num_cores * info.num_subcores
    lanes = info.num_lanes
    per = ((total + nw - 1) // nw + lanes - 1) // lanes * lanes
    table = gt_boxes.reshape(nb * m * 6)
    gathered = _make_sc_gather(total, per, nb * m * 6)(
        table, gidx.reshape(total)
    )
    gt_rois = jnp.transpose(gathered.reshape(6, nb, n), (1, 2, 0))
    return labels, all_rois, gt_rois


# speedup vs baseline: 1.0526x; 1.0526x over previous
import functools

import jax
import jax.numpy as jnp
from jax import lax
from jax.experimental import pallas as pl
from jax.experimental.pallas import tpu as pltpu
from jax.experimental.pallas import tpu_sc as plsc

FG_THRESHOLD = 0.5


def _tc_body(roist_ref, gt_ref, labels_ref, gidx_ref, *, nb, tp, m):
    miota = jax.lax.broadcasted_iota(jnp.int32, (m, tp), 0)
    for b in range(nb):
        g = gt_ref[b]
        rt = roist_ref[b]
        inter = None
        va = None
        vb = None
        for c in range(3):
            blo = rt[1 + c : 2 + c, :]
            bhi = rt[4 + c : 5 + c, :]
            glo = g[:, c : c + 1]
            ghi = g[:, 3 + c : 4 + c]
            d = jnp.maximum(jnp.minimum(bhi, ghi) - jnp.maximum(blo, glo), 0.0)
            inter = d if inter is None else inter * d
            sa = jnp.maximum(bhi - blo, 0.0)
            va = sa if va is None else va * sa
            sb = jnp.maximum(ghi - glo, 0.0)
            vb = sb if vb is None else vb * sb
        union = va + vb - inter
        iou = inter / union
        mx = jnp.max(iou, axis=0, keepdims=True)
        labels_ref[b] = (mx[0] >= FG_THRESHOLD).astype(jnp.int32)
        sel = jnp.where(iou == mx, miota, m)
        gidx_ref[b] = jnp.min(sel, axis=0) + b * m


def _make_sc_gather(total, per, tab_words):
    mesh = plsc.VectorSubcoreMesh(core_axis_name="c", subcore_axis_name="s")
    info = plsc.get_sparse_core_info()
    nc = info.num_cores
    lanes = info.num_lanes

    @functools.partial(
        pl.kernel,
        mesh=mesh,
        out_type=jax.ShapeDtypeStruct((total * 6,), jnp.float32),
        scratch_types=[
            pltpu.VMEM((per,), jnp.int32),
            pltpu.VMEM((per * 6,), jnp.float32),
            pltpu.VMEM((tab_words,), jnp.float32),
        ],
        compiler_params=pltpu.CompilerParams(
            use_tc_tiling_on_sc=False, needs_layout_passes=False
        ),
    )
    def sc_gather(table_hbm, idx_hbm, out_hbm, idx_v, rows_v, tab_v):
        wid = lax.axis_index("s") * nc + lax.axis_index("c")
        base = jnp.minimum(wid * per, total - per)
        pltpu.sync_copy(table_hbm, tab_v)
        pltpu.sync_copy(idx_hbm.at[pl.ds(base, per)], idx_v)
        lane_iota = jax.lax.iota(jnp.int32, lanes)

        def body(i, carry):
            for u in range(4):
                j0 = i * (4 * lanes) + u * lanes
                addr = idx_v[pl.ds(j0, lanes)] * 6
                dst = lane_iota + j0
                for c in range(6):
                    vals = plsc.load_gather(tab_v, [addr + c])
                    plsc.store_scatter(rows_v, [dst + c * per], vals)
            return carry

        lax.fori_loop(0, per // (4 * lanes), body, 0)
        for c in range(6):
            pltpu.sync_copy(
                rows_v.at[pl.ds(c * per, per)],
                out_hbm.at[pl.ds(c * total + base, per)],
            )

    return sc_gather


def kernel(all_rois, gt_boxes, gt_labels, is_sample):
    nb, n, _ = all_rois.shape
    m = gt_boxes.shape[1]
    tp = 2048
    rois_t = jnp.swapaxes(all_rois, 1, 2)
    labels, gidx = pl.pallas_call(
        functools.partial(_tc_body, nb=nb, tp=tp, m=m),
        grid=(pl.cdiv(n, tp),),
        in_specs=[
            pl.BlockSpec((nb, 7, tp), lambda i: (0, 0, i)),
            pl.BlockSpec((nb, m, 6), lambda i: (0, 0, 0)),
        ],
        out_specs=[
            pl.BlockSpec((nb, tp), lambda i: (0, i)),
            pl.BlockSpec((nb, tp), lambda i: (0, i)),
        ],
        out_shape=[
            jax.ShapeDtypeStruct((nb, n), jnp.int32),
            jax.ShapeDtypeStruct((nb, n), jnp.int32),
        ],
        compiler_params=pltpu.CompilerParams(
            dimension_semantics=("arbitrary",),
        ),
    )(rois_t, gt_boxes)

    total = nb * n
    info = plsc.get_sparse_core_info()
    nw = info.num_cores * info.num_subcores
    lanes = info.num_lanes
    per = ((total + nw - 1) // nw + 4 * lanes - 1) // (4 * lanes) * (4 * lanes)
    table = gt_boxes.reshape(nb * m * 6)
    gathered = _make_sc_gather(total, per, nb * m * 6)(
        table, gidx.reshape(total)
    )
    gt_rois = jnp.transpose(gathered.reshape(6, nb, n), (1, 2, 0))
    return labels, all_rois, gt_rois


# speedup vs baseline: 1.0570x; 1.0042x over previous
import functools

import jax
import jax.numpy as jnp
from jax import lax
from jax.experimental import pallas as pl
from jax.experimental.pallas import tpu as pltpu
from jax.experimental.pallas import tpu_sc as plsc

FG_THRESHOLD = 0.5


def _tc_body(roist_ref, gt_ref, labels_ref, gidx_ref, *, nb, tp, m):
    miota = jax.lax.broadcasted_iota(jnp.int32, (m, tp), 0)
    for b in range(nb):
        g = gt_ref[b]
        rt = roist_ref[b]
        inter = None
        va = None
        vb = None
        for c in range(3):
            blo = rt[1 + c : 2 + c, :]
            bhi = rt[4 + c : 5 + c, :]
            glo = g[:, c : c + 1]
            ghi = g[:, 3 + c : 4 + c]
            d = jnp.maximum(jnp.minimum(bhi, ghi) - jnp.maximum(blo, glo), 0.0)
            inter = d if inter is None else inter * d
            sa = jnp.maximum(bhi - blo, 0.0)
            va = sa if va is None else va * sa
            sb = jnp.maximum(ghi - glo, 0.0)
            vb = sb if vb is None else vb * sb
        union = va + vb - inter
        iou = inter / union
        mx = jnp.max(iou, axis=0, keepdims=True)
        labels_ref[b] = (mx[0] >= FG_THRESHOLD).astype(jnp.int32)
        sel = jnp.where(iou == mx, miota, m)
        gidx_ref[b] = jnp.min(sel, axis=0) + b * m


def _make_sc_gather(total, per, tab_words):
    mesh = plsc.VectorSubcoreMesh(core_axis_name="c", subcore_axis_name="s")
    info = plsc.get_sparse_core_info()
    nc = info.num_cores
    lanes = info.num_lanes

    @functools.partial(
        pl.kernel,
        mesh=mesh,
        out_type=jax.ShapeDtypeStruct((total * 6,), jnp.float32),
        scratch_types=[
            pltpu.VMEM((per,), jnp.int32),
            pltpu.VMEM((per * 6,), jnp.float32),
            pltpu.VMEM((tab_words,), jnp.float32),
        ],
        compiler_params=pltpu.CompilerParams(
            use_tc_tiling_on_sc=False, needs_layout_passes=False
        ),
    )
    def sc_gather(table_hbm, idx_hbm, out_hbm, idx_v, rows_v, tab_v):
        wid = lax.axis_index("s") * nc + lax.axis_index("c")
        base = jnp.minimum(wid * per, total - per)
        pltpu.sync_copy(table_hbm, tab_v)
        pltpu.sync_copy(idx_hbm.at[pl.ds(base, per)], idx_v)
        lane_iota = jax.lax.iota(jnp.int32, lanes)

        def body(i, carry):
            j0 = i * lanes
            addr = idx_v[pl.ds(j0, lanes)] * 6
            dst = lane_iota + j0
            for c in range(6):
                vals = plsc.load_gather(tab_v, [addr + c])
                plsc.store_scatter(rows_v, [dst + c * per], vals)
            return carry

        lax.fori_loop(0, per // lanes, body, 0)
        for c in range(6):
            pltpu.sync_copy(
                rows_v.at[pl.ds(c * per, per)],
                out_hbm.at[pl.ds(c * total + base, per)],
            )

    return sc_gather


def kernel(all_rois, gt_boxes, gt_labels, is_sample):
    nb, n, _ = all_rois.shape
    m = gt_boxes.shape[1]
    tp = 2048
    rois_t = jnp.swapaxes(all_rois, 1, 2)
    labels, gidx = pl.pallas_call(
        functools.partial(_tc_body, nb=nb, tp=tp, m=m),
        grid=(pl.cdiv(n, tp),),
        in_specs=[
            pl.BlockSpec((nb, 7, tp), lambda i: (0, 0, i)),
            pl.BlockSpec((nb, m, 6), lambda i: (0, 0, 0)),
        ],
        out_specs=[
            pl.BlockSpec((nb, tp), lambda i: (0, i)),
            pl.BlockSpec((nb, tp), lambda i: (0, i)),
        ],
        out_shape=[
            jax.ShapeDtypeStruct((nb, n), jnp.int32),
            jax.ShapeDtypeStruct((nb, n), jnp.int32),
        ],
        compiler_params=pltpu.CompilerParams(
            dimension_semantics=("arbitrary",),
        ),
    )(rois_t, gt_boxes)

    total = nb * n
    info = plsc.get_sparse_core_info()
    nw = info.num_cores * info.num_subcores
    lanes = info.num_lanes
    per = ((total + nw - 1) // nw + lanes - 1) // lanes * lanes
    table = gt_boxes.reshape(nb * m * 6)
    gathered = _make_sc_gather(total, per, nb * m * 6)(
        table, gidx.reshape(total)
    )
    gt_rois = jnp.transpose(gathered.reshape(6, nb, n), (1, 2, 0))
    return labels, all_rois, gt_rois
